# baseline (speedup 1.0000x reference)
import functools

import jax
import jax.numpy as jnp
from jax import lax
from jax.experimental import pallas as pl
from jax.experimental.pallas import tpu as pltpu
from jax.experimental.pallas import tpu_sc as plsc

_N = 10000
_E = 320000
_D = 128
_R = 3

_NC = 2
_NS = 16
_NW = _NC * _NS
_EPW = _E // _NW
_CHUNK = 104
_NCHUNK = 98
_EPWP = _NCHUNK * _CHUNK
_PAD = _EPWP - _EPW
_NP = 10112
_RPT = _NP // _NS
_NB = 2


def _segment_sum_sc(h, src3, dst3, zeros):
    mesh = plsc.VectorSubcoreMesh(
        core_axis_name="c", subcore_axis_name="s",
        num_cores=_NC, num_subcores=_NS)

    @functools.partial(
        pl.kernel,
        out_type=jax.ShapeDtypeStruct((_NC, _NP, _D), jnp.float32),
        mesh=mesh,
        scratch_types=[
            pltpu.VMEM((_EPWP,), jnp.int32),
            pltpu.VMEM((_NCHUNK, _CHUNK), jnp.int32),
            pltpu.VMEM((_NB, _CHUNK, _D), jnp.float32),
            pltpu.VMEM_SHARED((_NP, _D), jnp.float32),
            [pltpu.SemaphoreType.DMA] * _NB,
        ],
    )
    def seg_kernel(h_hbm, src_hbm, dst_hbm, z_hbm, out_hbm,
                   src_v, dst_v, rows, acc_sh, sems):
        c = lax.axis_index("c")
        s = lax.axis_index("s")
        w = s * _NC + c
        pltpu.sync_copy(z_hbm.at[pl.ds(s * _RPT, _RPT)],
                        acc_sh.at[pl.ds(s * _RPT, _RPT)])
        pltpu.sync_copy(src_hbm.at[pl.ds(w * _EPWP, _EPWP)], src_v)
        pltpu.sync_copy(dst_hbm.at[w], dst_v)
        plsc.subcore_barrier()

        def outer(jo, carry):
            for b in range(_NB):
                j = jo * _NB + b

                @pl.when(jo > 0)
                def _():
                    pltpu.make_async_copy(
                        rows.at[b], acc_sh.at[dst_v.at[j]], sems[b]).wait()

                pltpu.sync_copy(h_hbm.at[src_v.at[pl.ds(j * _CHUNK, _CHUNK)]],
                                rows.at[b])
                pltpu.async_copy(rows.at[b], acc_sh.at[dst_v.at[j]],
                                 sems[b], add=True)
            return carry

        lax.fori_loop(0, _NCHUNK // _NB, outer, 0)
        for b in range(_NB):
            pltpu.make_async_copy(
                rows.at[b], acc_sh.at[dst_v.at[_NCHUNK - _NB + b]],
                sems[b]).wait()
        plsc.subcore_barrier()
        pltpu.sync_copy(acc_sh.at[pl.ds(s * _RPT, _RPT)],
                        out_hbm.at[c, pl.ds(s * _RPT, _RPT)])

    return seg_kernel(h, src3, dst3, zeros)


def _mlp_body(x_ref, w1_ref, b1_ref, w2_ref, b2_ref, o_ref):
    h = jnp.dot(x_ref[...], w1_ref[...], preferred_element_type=jnp.float32)
    h = jnp.maximum(h + b1_ref[...], 0.0)
    h = jnp.dot(h, w2_ref[...], preferred_element_type=jnp.float32)
    o_ref[...] = jnp.maximum(h + b2_ref[...], 0.0)


def _mlp(x, w1t, b1, w2t, b2):
    return pl.pallas_call(
        _mlp_body,
        out_shape=jax.ShapeDtypeStruct((_N, _D), jnp.float32),
    )(x, w1t, b1.reshape(1, _D), w2t, b2.reshape(1, _D))


def _conv_body(h_ref, a_ref, wrt_ref, br_ref, wot_ref, o_ref):
    agg = a_ref[0, :_N] + a_ref[1, :_N]
    conv = jnp.dot(agg, wrt_ref[...], preferred_element_type=jnp.float32)
    conv = conv + br_ref[...]
    conv = conv + jnp.dot(h_ref[...], wot_ref[...],
                          preferred_element_type=jnp.float32)
    o_ref[...] = h_ref[...] + jnp.maximum(conv, 0.0)


def _conv(h, aggp, wrt, br, wot):
    return pl.pallas_call(
        _conv_body,
        out_shape=jax.ShapeDtypeStruct((_N, _D), jnp.float32),
    )(h, aggp, wrt, br.reshape(1, _D), wot)


def _conv_out_body(h_ref, a_ref, wrt_ref, br_ref, wot_ref,
                   wout_ref, bout_ref, o_ref):
    agg = a_ref[0, :_N] + a_ref[1, :_N]
    conv = jnp.dot(agg, wrt_ref[...], preferred_element_type=jnp.float32)
    conv = conv + br_ref[...]
    conv = conv + jnp.dot(h_ref[...], wot_ref[...],
                          preferred_element_type=jnp.float32)
    hn = h_ref[...] + jnp.maximum(conv, 0.0)
    o_ref[...] = jnp.dot(hn, wout_ref[...],
                         preferred_element_type=jnp.float32) + bout_ref[...]


def _conv_out(h, aggp, wrt, br, wot, woutt, bout):
    return pl.pallas_call(
        _conv_out_body,
        out_shape=jax.ShapeDtypeStruct((_N, woutt.shape[1]), jnp.float32),
    )(h, aggp, wrt, br.reshape(1, _D), wot,
      woutt, bout.reshape(1, woutt.shape[1]))


def kernel(x, edge_index, batch, W1, b1, W2, b2, Wr, br, Wo, Wout, bout):
    del batch
    src1 = jnp.pad(edge_index[0].reshape(_NW, _EPW),
                   ((0, 0), (0, _PAD))).reshape(-1)
    dst3 = jnp.pad(edge_index[1].reshape(_NW, _EPW),
                   ((0, 0), (0, _PAD)),
                   constant_values=_NP - 1).reshape(_NW, _NCHUNK, _CHUNK)
    zeros = jnp.zeros((_NP, _D), jnp.float32)

    h = _mlp(x, W1.T, b1, W2.T, b2)
    for i in range(_R):
        aggp = _segment_sum_sc(h, src1, dst3, zeros)
        if i < _R - 1:
            h = _conv(h, aggp, Wr[i].T, br[i], Wo[i].T)
        else:
            h = _conv_out(h, aggp, Wr[i].T, br[i], Wo[i].T, Wout.T, bout)
    return h


# speedup vs baseline: 2.3523x; 2.3523x over previous
import functools

import jax
import jax.numpy as jnp
from jax import lax
from jax.experimental import pallas as pl
from jax.experimental.pallas import tpu as pltpu
from jax.experimental.pallas import tpu_sc as plsc

_N = 10000
_E = 320000
_D = 128
_R = 3

_NC = 2
_NS = 16
_NW = _NC * _NS
_EPW = _E // _NW
_CHUNK = 104
_NCHUNK = 98
_EPWP = _NCHUNK * _CHUNK
_PAD = _EPWP - _EPW
_NP = 10112
_RPT = _NP // _NS
_NB = 2


def _segment_sum_sc(h, src3, dst3, zeros):
    mesh = plsc.VectorSubcoreMesh(
        core_axis_name="c", subcore_axis_name="s",
        num_cores=_NC, num_subcores=_NS)

    @functools.partial(
        pl.kernel,
        out_type=jax.ShapeDtypeStruct((_NC, _NP, _D), jnp.float32),
        mesh=mesh,
        scratch_types=[
            pltpu.VMEM((_EPWP,), jnp.int32),
            pltpu.VMEM((_NCHUNK, _CHUNK), jnp.int32),
            pltpu.VMEM((_NB, _CHUNK, _D), jnp.float32),
            pltpu.VMEM_SHARED((_NP, _D), jnp.float32),
            [pltpu.SemaphoreType.DMA] * _NB,
        ],
    )
    def seg_kernel(h_hbm, src_hbm, dst_hbm, z_hbm, out_hbm,
                   src_v, dst_v, rows, acc_sh, sems):
        c = lax.axis_index("c")
        s = lax.axis_index("s")
        w = s * _NC + c
        pltpu.sync_copy(z_hbm.at[pl.ds(s * _RPT, _RPT)],
                        acc_sh.at[pl.ds(s * _RPT, _RPT)])
        pltpu.sync_copy(src_hbm.at[pl.ds(w * _EPWP, _EPWP)], src_v)
        pltpu.sync_copy(dst_hbm.at[w], dst_v)
        plsc.subcore_barrier()

        def outer(jo, carry):
            for b in range(_NB):
                j = jo * _NB + b

                @pl.when(jo > 0)
                def _():
                    pltpu.make_async_copy(
                        rows.at[b], acc_sh.at[dst_v.at[j]], sems[b]).wait()

                pltpu.sync_copy(h_hbm.at[src_v.at[pl.ds(j * _CHUNK, _CHUNK)]],
                                rows.at[b])
                pltpu.async_copy(rows.at[b], acc_sh.at[dst_v.at[j]],
                                 sems[b], add=True)
            return carry

        lax.fori_loop(0, _NCHUNK // _NB, outer, 0)
        for b in range(_NB):
            pltpu.make_async_copy(
                rows.at[b], acc_sh.at[dst_v.at[_NCHUNK - _NB + b]],
                sems[b]).wait()
        plsc.subcore_barrier()
        pltpu.sync_copy(acc_sh.at[pl.ds(s * _RPT, _RPT)],
                        out_hbm.at[c, pl.ds(s * _RPT, _RPT)])

    return seg_kernel(h, src3, dst3, zeros)


def _mlp_body(x_ref, w1_ref, b1_ref, w2_ref, b2_ref, o_ref):
    h = jnp.dot(x_ref[...], w1_ref[...], preferred_element_type=jnp.float32)
    h = jnp.maximum(h + b1_ref[...], 0.0)
    h = jnp.dot(h, w2_ref[...], preferred_element_type=jnp.float32)
    o_ref[:_N] = jnp.maximum(h + b2_ref[...], 0.0)
    o_ref[_N:] = jnp.zeros((_NP - _N, _D), jnp.float32)


def _mlp(x, w1t, b1, w2t, b2):
    return pl.pallas_call(
        _mlp_body,
        out_shape=jax.ShapeDtypeStruct((_NP, _D), jnp.float32),
    )(x, w1t, b1.reshape(1, _D), w2t, b2.reshape(1, _D))


def _conv_body(h_ref, a_ref, wrt_ref, br_ref, wot_ref, o_ref):
    agg = a_ref[0, :_N] + a_ref[1, :_N]
    conv = jnp.dot(agg, wrt_ref[...], preferred_element_type=jnp.float32)
    conv = conv + br_ref[...]
    conv = conv + jnp.dot(h_ref[:_N], wot_ref[...],
                          preferred_element_type=jnp.float32)
    o_ref[:_N] = h_ref[:_N] + jnp.maximum(conv, 0.0)
    o_ref[_N:] = jnp.zeros((_NP - _N, _D), jnp.float32)


def _conv(h, aggp, wrt, br, wot):
    return pl.pallas_call(
        _conv_body,
        out_shape=jax.ShapeDtypeStruct((_NP, _D), jnp.float32),
    )(h, aggp, wrt, br.reshape(1, _D), wot)


def _conv_out_body(h_ref, a_ref, wrt_ref, br_ref, wot_ref,
                   wout_ref, bout_ref, o_ref):
    agg = a_ref[0, :_N] + a_ref[1, :_N]
    conv = jnp.dot(agg, wrt_ref[...], preferred_element_type=jnp.float32)
    conv = conv + br_ref[...]
    conv = conv + jnp.dot(h_ref[:_N], wot_ref[...],
                          preferred_element_type=jnp.float32)
    hn = h_ref[:_N] + jnp.maximum(conv, 0.0)
    o_ref[...] = jnp.dot(hn, wout_ref[...],
                         preferred_element_type=jnp.float32) + bout_ref[...]


def _conv_out(h, aggp, wrt, br, wot, woutt, bout):
    return pl.pallas_call(
        _conv_out_body,
        out_shape=jax.ShapeDtypeStruct((_N, woutt.shape[1]), jnp.float32),
    )(h, aggp, wrt, br.reshape(1, _D), wot,
      woutt, bout.reshape(1, woutt.shape[1]))


def kernel(x, edge_index, batch, W1, b1, W2, b2, Wr, br, Wo, Wout, bout):
    del batch
    pad_src = _N + (jnp.arange(_PAD, dtype=jnp.int32) % (_NP - _N))
    src1 = jnp.concatenate(
        [edge_index[0].reshape(_NW, _EPW),
         jnp.broadcast_to(pad_src, (_NW, _PAD))], axis=1).reshape(-1)
    pad_dst = (jnp.arange(_NW * _PAD, dtype=jnp.int32) * 131) % _NP
    dst3 = jnp.concatenate(
        [edge_index[1].reshape(_NW, _EPW),
         pad_dst.reshape(_NW, _PAD)], axis=1).reshape(_NW, _NCHUNK, _CHUNK)
    zeros = jnp.zeros((_NP, _D), jnp.float32)

    h = _mlp(x, W1.T, b1, W2.T, b2)
    for i in range(_R):
        aggp = _segment_sum_sc(h, src1, dst3, zeros)
        if i < _R - 1:
            h = _conv(h, aggp, Wr[i].T, br[i], Wo[i].T)
        else:
            h = _conv_out(h, aggp, Wr[i].T, br[i], Wo[i].T, Wout.T, bout)
    return h


# speedup vs baseline: 2.3571x; 1.0021x over previous
import functools

import jax
import jax.numpy as jnp
from jax import lax
from jax.experimental import pallas as pl
from jax.experimental.pallas import tpu as pltpu
from jax.experimental.pallas import tpu_sc as plsc

_N = 10000
_E = 320000
_D = 128
_R = 3

_NC = 2
_NS = 16
_NW = _NC * _NS
_EPW = _E // _NW
_CHUNK = 104
_NCHUNK = 98
_EPWP = _NCHUNK * _CHUNK
_PAD = _EPWP - _EPW
_NP = 10112
_RPT = _NP // _NS
_NB = 2


def _segment_sum_sc(h, src3, dst3, zeros):
    mesh = plsc.VectorSubcoreMesh(
        core_axis_name="c", subcore_axis_name="s",
        num_cores=_NC, num_subcores=_NS)

    @functools.partial(
        pl.kernel,
        out_type=jax.ShapeDtypeStruct((_NC, _NP, _D), jnp.float32),
        mesh=mesh,
        scratch_types=[
            pltpu.VMEM((_EPWP,), jnp.int32),
            pltpu.VMEM((_NCHUNK, _CHUNK), jnp.int32),
            pltpu.VMEM((_NB, _CHUNK, _D), jnp.float32),
            pltpu.VMEM_SHARED((_NP, _D), jnp.float32),
            [pltpu.SemaphoreType.DMA] * _NB,
        ],
    )
    def seg_kernel(h_hbm, src_hbm, dst_hbm, z_hbm, out_hbm,
                   src_v, dst_v, rows, acc_sh, sems):
        c = lax.axis_index("c")
        s = lax.axis_index("s")
        w = s * _NC + c
        pltpu.sync_copy(z_hbm.at[pl.ds(s * _RPT, _RPT)],
                        acc_sh.at[pl.ds(s * _RPT, _RPT)])
        pltpu.sync_copy(src_hbm.at[pl.ds(w * _EPWP, _EPWP)], src_v)
        pltpu.sync_copy(dst_hbm.at[w], dst_v)
        plsc.subcore_barrier()

        pltpu.async_copy(h_hbm.at[src_v.at[pl.ds(0, _CHUNK)]],
                         rows.at[0], sems[0])

        def outer(jo, carry):
            for b in range(_NB):
                j = jo * _NB + b
                pltpu.make_async_copy(
                    h_hbm.at[src_v.at[pl.ds(j * _CHUNK, _CHUNK)]],
                    rows.at[b], sems[b]).wait()

                @pl.when(j < _NCHUNK - 1)
                def _():
                    jn = j + 1
                    pltpu.async_copy(
                        h_hbm.at[src_v.at[pl.ds(jn * _CHUNK, _CHUNK)]],
                        rows.at[1 - b], sems[1 - b])

                pltpu.sync_copy(rows.at[b], acc_sh.at[dst_v.at[j]], add=True)
            return carry

        lax.fori_loop(0, _NCHUNK // _NB, outer, 0)
        plsc.subcore_barrier()
        pltpu.sync_copy(acc_sh.at[pl.ds(s * _RPT, _RPT)],
                        out_hbm.at[c, pl.ds(s * _RPT, _RPT)])

    return seg_kernel(h, src3, dst3, zeros)


def _mlp_body(x_ref, w1_ref, b1_ref, w2_ref, b2_ref, o_ref):
    h = jnp.dot(x_ref[...], w1_ref[...], preferred_element_type=jnp.float32)
    h = jnp.maximum(h + b1_ref[...], 0.0)
    h = jnp.dot(h, w2_ref[...], preferred_element_type=jnp.float32)
    o_ref[:_N] = jnp.maximum(h + b2_ref[...], 0.0)
    o_ref[_N:] = jnp.zeros((_NP - _N, _D), jnp.float32)


def _mlp(x, w1t, b1, w2t, b2):
    return pl.pallas_call(
        _mlp_body,
        out_shape=jax.ShapeDtypeStruct((_NP, _D), jnp.float32),
    )(x, w1t, b1.reshape(1, _D), w2t, b2.reshape(1, _D))


def _conv_body(h_ref, a_ref, wrt_ref, br_ref, wot_ref, o_ref):
    agg = a_ref[0, :_N] + a_ref[1, :_N]
    conv = jnp.dot(agg, wrt_ref[...], preferred_element_type=jnp.float32)
    conv = conv + br_ref[...]
    conv = conv + jnp.dot(h_ref[:_N], wot_ref[...],
                          preferred_element_type=jnp.float32)
    o_ref[:_N] = h_ref[:_N] + jnp.maximum(conv, 0.0)
    o_ref[_N:] = jnp.zeros((_NP - _N, _D), jnp.float32)


def _conv(h, aggp, wrt, br, wot):
    return pl.pallas_call(
        _conv_body,
        out_shape=jax.ShapeDtypeStruct((_NP, _D), jnp.float32),
    )(h, aggp, wrt, br.reshape(1, _D), wot)


def _conv_out_body(h_ref, a_ref, wrt_ref, br_ref, wot_ref,
                   wout_ref, bout_ref, o_ref):
    agg = a_ref[0, :_N] + a_ref[1, :_N]
    conv = jnp.dot(agg, wrt_ref[...], preferred_element_type=jnp.float32)
    conv = conv + br_ref[...]
    conv = conv + jnp.dot(h_ref[:_N], wot_ref[...],
                          preferred_element_type=jnp.float32)
    hn = h_ref[:_N] + jnp.maximum(conv, 0.0)
    o_ref[...] = jnp.dot(hn, wout_ref[...],
                         preferred_element_type=jnp.float32) + bout_ref[...]


def _conv_out(h, aggp, wrt, br, wot, woutt, bout):
    return pl.pallas_call(
        _conv_out_body,
        out_shape=jax.ShapeDtypeStruct((_N, woutt.shape[1]), jnp.float32),
    )(h, aggp, wrt, br.reshape(1, _D), wot,
      woutt, bout.reshape(1, woutt.shape[1]))


def kernel(x, edge_index, batch, W1, b1, W2, b2, Wr, br, Wo, Wout, bout):
    del batch
    pad_src = _N + (jnp.arange(_PAD, dtype=jnp.int32) % (_NP - _N))
    src1 = jnp.concatenate(
        [edge_index[0].reshape(_NW, _EPW),
         jnp.broadcast_to(pad_src, (_NW, _PAD))], axis=1).reshape(-1)
    pad_dst = (jnp.arange(_NW * _PAD, dtype=jnp.int32) * 131) % _NP
    dst3 = jnp.concatenate(
        [edge_index[1].reshape(_NW, _EPW),
         pad_dst.reshape(_NW, _PAD)], axis=1).reshape(_NW, _NCHUNK, _CHUNK)
    zeros = jnp.zeros((_NP, _D), jnp.float32)

    h = _mlp(x, W1.T, b1, W2.T, b2)
    for i in range(_R):
        aggp = _segment_sum_sc(h, src1, dst3, zeros)
        if i < _R - 1:
            h = _conv(h, aggp, Wr[i].T, br[i], Wo[i].T)
        else:
            h = _conv_out(h, aggp, Wr[i].T, br[i], Wo[i].T, Wout.T, bout)
    return h
